# baseline (speedup 1.0000x reference)
import functools

import jax
import jax.numpy as jnp
from jax import lax
from jax.experimental import pallas as pl
from jax.experimental.pallas import tpu as pltpu
from jax.experimental.pallas import tpu_sc as plsc

_NC = 2
_NS = 16
_NW = _NC * _NS
_CH = 128


def _widen_rows_tc(emb, rows_per_block):
    n, d = emb.shape

    def body(x_ref, o_ref):
        o_ref[:, :d] = x_ref[...]

    return pl.pallas_call(
        body,
        grid=(n // rows_per_block,),
        in_specs=[pl.BlockSpec((rows_per_block, d), lambda i: (i, 0))],
        out_specs=pl.BlockSpec((rows_per_block, 128), lambda i: (i, 0)),
        out_shape=jax.ShapeDtypeStruct((n, 128), emb.dtype),
    )(emb)


@functools.partial(jax.jit, static_argnums=(2, 3))
def _sc_gather(emb128, idx1, n_total, n_ch):
    mesh = plsc.VectorSubcoreMesh(
        core_axis_name="c", subcore_axis_name="s",
        num_cores=_NC, num_subcores=_NS)

    @functools.partial(
        pl.kernel,
        out_type=jax.ShapeDtypeStruct((n_total, 128), jnp.float32),
        mesh=mesh,
        scratch_types=[
            pltpu.VMEM((n_ch * _CH,), jnp.int32),
            pltpu.VMEM((_CH, 128), jnp.float32),
            pltpu.VMEM((_CH, 128), jnp.float32),
            pltpu.SemaphoreType.DMA,
            pltpu.SemaphoreType.DMA,
        ],
        compiler_params=pltpu.CompilerParams(use_tc_tiling_on_sc=True),
    )
    def k(emb_hbm, idx_hbm, out_hbm, idx_v, buf0, buf1, sem0, sem1):
        wid = lax.axis_index("s") * _NC + lax.axis_index("c")
        base = wid * (n_ch * _CH)
        pltpu.sync_copy(idx_hbm.at[pl.ds(base, n_ch * _CH)], idx_v)

        def issue(j, buf, sem):
            pltpu.async_copy(
                emb_hbm.at[idx_v.at[pl.ds(j * _CH, _CH)]], buf, sem)

        def drain(buf, sem):
            pltpu.make_async_copy(emb_hbm.at[pl.ds(0, _CH)], buf, sem).wait()

        def writeback(j, buf):
            pltpu.sync_copy(buf, out_hbm.at[pl.ds(base + j * _CH, _CH)])

        issue(0, buf0, sem0)
        issue(1, buf1, sem1)

        def body(i, carry):
            j = 2 * i
            drain(buf0, sem0)
            writeback(j, buf0)
            issue(j + 2, buf0, sem0)
            drain(buf1, sem1)
            writeback(j + 1, buf1)
            issue(j + 3, buf1, sem1)
            return carry

        lax.fori_loop(0, n_ch // 2 - 1, body, 0)
        j_last = n_ch - 2
        drain(buf0, sem0)
        writeback(j_last, buf0)
        drain(buf1, sem1)
        writeback(j_last + 1, buf1)

    return k(emb128, idx1)


def kernel(x, embeddings):
    b, t = x.shape
    n_states, d = embeddings.shape
    n_total = b * t
    n_ch = n_total // (_NW * _CH)
    idx1 = x.reshape(n_total).astype(jnp.int32)
    emb128 = _widen_rows_tc(embeddings, 8000)
    out128 = _sc_gather(emb128, idx1, n_total, n_ch)
    return out128[:, :d].reshape(b, t, d)


# speedup vs baseline: 1.0002x; 1.0002x over previous
import functools

import jax
import jax.numpy as jnp
from jax import lax
from jax.experimental import pallas as pl
from jax.experimental.pallas import tpu as pltpu
from jax.experimental.pallas import tpu_sc as plsc

_NC = 2
_NS = 16
_NW = _NC * _NS
_CH = 128


def _widen_rows_tc(emb, rows_per_block):
    n, d = emb.shape

    def body(x_ref, o_ref):
        o_ref[...] = jnp.pad(x_ref[...], ((0, 0), (0, 128 - d)))

    return pl.pallas_call(
        body,
        grid=(n // rows_per_block,),
        in_specs=[pl.BlockSpec((rows_per_block, d), lambda i: (i, 0))],
        out_specs=pl.BlockSpec((rows_per_block, 128), lambda i: (i, 0)),
        out_shape=jax.ShapeDtypeStruct((n, 128), emb.dtype),
    )(emb)


@functools.partial(jax.jit, static_argnums=(2, 3))
def _sc_gather(emb128, idx1, n_total, n_ch):
    mesh = plsc.VectorSubcoreMesh(
        core_axis_name="c", subcore_axis_name="s",
        num_cores=_NC, num_subcores=_NS)

    @functools.partial(
        pl.kernel,
        out_type=jax.ShapeDtypeStruct((n_total, 128), jnp.float32),
        mesh=mesh,
        scratch_types=[
            pltpu.VMEM((n_ch * _CH,), jnp.int32),
            pltpu.VMEM((_CH, 128), jnp.float32),
            pltpu.VMEM((_CH, 128), jnp.float32),
            pltpu.SemaphoreType.DMA,
            pltpu.SemaphoreType.DMA,
        ],
        compiler_params=pltpu.CompilerParams(use_tc_tiling_on_sc=True),
    )
    def k(emb_hbm, idx_hbm, out_hbm, idx_v, buf0, buf1, sem0, sem1):
        wid = lax.axis_index("s") * _NC + lax.axis_index("c")
        base = wid * (n_ch * _CH)
        pltpu.sync_copy(idx_hbm.at[pl.ds(base, n_ch * _CH)], idx_v)

        def issue(j, buf, sem):
            pltpu.async_copy(
                emb_hbm.at[idx_v.at[pl.ds(j * _CH, _CH)]], buf, sem)

        def drain(buf, sem):
            pltpu.make_async_copy(emb_hbm.at[pl.ds(0, _CH)], buf, sem).wait()

        def writeback(j, buf):
            pltpu.sync_copy(buf, out_hbm.at[pl.ds(base + j * _CH, _CH)])

        issue(0, buf0, sem0)
        issue(1, buf1, sem1)

        def body(i, carry):
            j = 2 * i
            drain(buf0, sem0)
            writeback(j, buf0)
            issue(j + 2, buf0, sem0)
            drain(buf1, sem1)
            writeback(j + 1, buf1)
            issue(j + 3, buf1, sem1)
            return carry

        lax.fori_loop(0, n_ch // 2 - 1, body, 0)
        j_last = n_ch - 2
        drain(buf0, sem0)
        writeback(j_last, buf0)
        drain(buf1, sem1)
        writeback(j_last + 1, buf1)

    return k(emb128, idx1)


def kernel(x, embeddings):
    b, t = x.shape
    n_states, d = embeddings.shape
    n_total = b * t
    n_ch = n_total // (_NW * _CH)
    idx1 = x.reshape(n_total).astype(jnp.int32)
    emb128 = _widen_rows_tc(embeddings, 8000)
    out128 = _sc_gather(emb128, idx1, n_total, n_ch)
    return out128[:, :d].reshape(b, t, d)


# speedup vs baseline: 1.1683x; 1.1681x over previous
import functools

import jax
import jax.numpy as jnp
from jax import lax
from jax.experimental import pallas as pl
from jax.experimental.pallas import tpu as pltpu
from jax.experimental.pallas import tpu_sc as plsc

_NC = 2
_NS = 16
_NW = _NC * _NS
_CH = 128


def _widen_rows_tc(emb, rows_per_block):
    n, d = emb.shape

    def body(x_ref, o_ref):
        o_ref[...] = jnp.pad(x_ref[...], ((0, 0), (0, 128 - d)))

    return pl.pallas_call(
        body,
        grid=(n // rows_per_block,),
        in_specs=[pl.BlockSpec((rows_per_block, d), lambda i: (i, 0))],
        out_specs=pl.BlockSpec((rows_per_block, 128), lambda i: (i, 0)),
        out_shape=jax.ShapeDtypeStruct((n, 128), emb.dtype),
    )(emb)


@functools.partial(jax.jit, static_argnums=(2, 3))
def _sc_gather(emb128, idx1, n_total, n_ch):
    mesh = plsc.VectorSubcoreMesh(
        core_axis_name="c", subcore_axis_name="s",
        num_cores=_NC, num_subcores=_NS)

    @functools.partial(
        pl.kernel,
        out_type=jax.ShapeDtypeStruct((n_total, 128), jnp.float32),
        mesh=mesh,
        scratch_types=[
            pltpu.VMEM((n_ch * _CH,), jnp.int32),
            pltpu.VMEM((_CH, 128), jnp.float32),
            pltpu.VMEM((_CH, 128), jnp.float32),
            pltpu.SemaphoreType.DMA,
            pltpu.SemaphoreType.DMA,
        ],
        compiler_params=pltpu.CompilerParams(use_tc_tiling_on_sc=True),
    )
    def k(emb_hbm, idx_hbm, out_hbm, idx_v, buf0, buf1, sem0, sem1):
        wid = lax.axis_index("s") * _NC + lax.axis_index("c")
        base = wid * (n_ch * _CH)
        pltpu.sync_copy(idx_hbm.at[pl.ds(base, n_ch * _CH)], idx_v)

        def issue(j, buf, sem):
            pltpu.async_copy(
                emb_hbm.at[idx_v.at[pl.ds(j * _CH, _CH)]], buf, sem)

        def drain(buf, sem):
            pltpu.make_async_copy(emb_hbm.at[pl.ds(0, _CH)], buf, sem).wait()

        def writeback(j, buf):
            pltpu.sync_copy(buf, out_hbm.at[pl.ds(base + j * _CH, _CH)])

        issue(0, buf0, sem0)
        issue(1, buf1, sem1)

        def body(i, carry):
            j = 2 * i
            drain(buf0, sem0)
            writeback(j, buf0)
            issue(j + 2, buf0, sem0)
            drain(buf1, sem1)
            writeback(j + 1, buf1)
            issue(j + 3, buf1, sem1)
            return carry

        lax.fori_loop(0, n_ch // 2 - 1, body, 0)
        j_last = n_ch - 2
        drain(buf0, sem0)
        writeback(j_last, buf0)
        drain(buf1, sem1)
        writeback(j_last + 1, buf1)

    return k(emb128, idx1)


def kernel(x, embeddings):
    b, t = x.shape
    n_states, d = embeddings.shape
    n_total = b * t
    n_ch = n_total // (_NW * _CH)
    idx1 = x.reshape(n_total).astype(jnp.int32)
    emb128 = jnp.pad(embeddings, ((0, 0), (0, 128 - d)))
    out128 = _sc_gather(emb128, idx1, n_total, n_ch)
    return out128[:, :d].reshape(b, t, d)
